# baseline (speedup 1.0000x reference)
import functools

import jax
import jax.numpy as jnp
from jax import lax
from jax.experimental import pallas as pl
from jax.experimental.pallas import tpu as pltpu
from jax.experimental.pallas import tpu_sc as plsc

MEM_DIM = 128
N_ID = 16384
_HALF = N_ID // 2

_NC = 2
_NS = 16
_NW = _NC * _NS
_CHUNK = 128


def _make_sc_gather(n):
    b_per_w = n // _NW
    n_chunks = b_per_w // _CHUNK

    def body(n_id_hbm, mem_hbm, lu_hbm, h_out, lu_out, idx_v, rows_v, lu_v,
             sem, sem_lu):
        wid = lax.axis_index("s") * _NC + lax.axis_index("c")
        base = wid * b_per_w
        pltpu.sync_copy(n_id_hbm.at[pl.ds(base, b_per_w)], idx_v)
        row_cps = []
        lu_cps = []
        for c in range(n_chunks):
            idx_c = idx_v.at[pl.ds(c * _CHUNK, _CHUNK)]
            row_cps.append(
                pltpu.async_copy(mem_hbm.at[idx_c], rows_v.at[c], sem))
            lu_cps.append(
                pltpu.async_copy(lu_hbm.at[idx_c], lu_v.at[c], sem_lu))
        for c in range(n_chunks):
            off = base + c * _CHUNK
            row_cps[c].wait()
            pltpu.sync_copy(rows_v.at[c], h_out.at[pl.ds(off, _CHUNK)])
            lu_cps[c].wait()
            pltpu.sync_copy(lu_v.at[c], lu_out.at[pl.ds(off, _CHUNK)])

    return functools.partial(
        pl.kernel,
        mesh=plsc.VectorSubcoreMesh(core_axis_name="c", subcore_axis_name="s"),
        out_type=[
            jax.ShapeDtypeStruct((n, MEM_DIM), jnp.float32),
            jax.ShapeDtypeStruct((n,), jnp.int32),
        ],
        scratch_types=[
            pltpu.VMEM((b_per_w,), jnp.int32),
            pltpu.VMEM((n_chunks, _CHUNK, MEM_DIM), jnp.float32),
            pltpu.VMEM((n_chunks, _CHUNK), jnp.int32),
            pltpu.SemaphoreType.DMA,
            pltpu.SemaphoreType.DMA,
        ],
    )(body)


_sc_gather_half = _make_sc_gather(_HALF)

_BLK = 4096
_BLKS_PER_HALF = _HALF // _BLK


def _gru_body(h_ref, w_ref, bih_ref, bhh_ref, out_ref):
    h = h_ref[...]
    w = w_ref[...]
    gh = jax.lax.dot_general(
        h, w, (((1,), (0,)), ((), ())),
        preferred_element_type=jnp.float32,
    ) + bhh_ref[...]
    bih = bih_ref[...]
    r = jax.nn.sigmoid(bih[:, :MEM_DIM] + gh[:, :MEM_DIM])
    z = jax.nn.sigmoid(bih[:, MEM_DIM:2 * MEM_DIM] + gh[:, MEM_DIM:2 * MEM_DIM])
    n = jnp.tanh(bih[:, 2 * MEM_DIM:] + r * gh[:, 2 * MEM_DIM:])
    out_ref[...] = (1.0 - z) * n + z * h


def _gru_body_alias(h_ref, w_ref, bih_ref, bhh_ref, prev_ref, out_ref):
    del prev_ref
    _gru_body(h_ref, w_ref, bih_ref, bhh_ref, out_ref)


def _gru_first(h, w_hh_t, b_ih, b_hh):
    return pl.pallas_call(
        _gru_body,
        grid=(_BLKS_PER_HALF,),
        in_specs=[
            pl.BlockSpec((_BLK, MEM_DIM), lambda i: (i, 0)),
            pl.BlockSpec((MEM_DIM, 3 * MEM_DIM), lambda i: (0, 0)),
            pl.BlockSpec((1, 3 * MEM_DIM), lambda i: (0, 0)),
            pl.BlockSpec((1, 3 * MEM_DIM), lambda i: (0, 0)),
        ],
        out_specs=pl.BlockSpec((_BLK, MEM_DIM), lambda i: (i, 0)),
        out_shape=jax.ShapeDtypeStruct((N_ID, MEM_DIM), jnp.float32),
    )(h, w_hh_t, b_ih, b_hh)


def _gru_second(h, w_hh_t, b_ih, b_hh, prev):
    return pl.pallas_call(
        _gru_body_alias,
        grid=(_BLKS_PER_HALF,),
        in_specs=[
            pl.BlockSpec((_BLK, MEM_DIM), lambda i: (i, 0)),
            pl.BlockSpec((MEM_DIM, 3 * MEM_DIM), lambda i: (0, 0)),
            pl.BlockSpec((1, 3 * MEM_DIM), lambda i: (0, 0)),
            pl.BlockSpec((1, 3 * MEM_DIM), lambda i: (0, 0)),
            pl.BlockSpec(memory_space=pl.ANY),
        ],
        out_specs=pl.BlockSpec(
            (_BLK, MEM_DIM), lambda i: (i + _BLKS_PER_HALF, 0)),
        out_shape=jax.ShapeDtypeStruct((N_ID, MEM_DIM), jnp.float32),
        input_output_aliases={4: 0},
    )(h, w_hh_t, b_ih, b_hh, prev)


@jax.jit
def kernel(n_id, memory, last_update, W_ih, W_hh, b_ih, b_hh):
    del W_ih
    n_id = n_id.astype(jnp.int32)
    lu32 = last_update.astype(jnp.int32)
    h0, lu0 = _sc_gather_half(n_id[:_HALF], memory, lu32)
    h1, lu1 = _sc_gather_half(n_id[_HALF:], memory, lu32)
    w_t = W_hh.T
    bih = b_ih.reshape(1, -1)
    bhh = b_hh.reshape(1, -1)
    out0 = _gru_first(h0, w_t, bih, bhh)
    mem_out = _gru_second(h1, w_t, bih, bhh, out0)
    lu_out = jnp.concatenate([lu0, lu1]).astype(last_update.dtype)
    return (mem_out, lu_out)


# speedup vs baseline: 1.0708x; 1.0708x over previous
import functools

import jax
import jax.numpy as jnp
from jax import lax
from jax.experimental import pallas as pl
from jax.experimental.pallas import tpu as pltpu
from jax.experimental.pallas import tpu_sc as plsc

MEM_DIM = 128
N_ID = 16384

_NC = 2
_NS = 16
_NW = _NC * _NS
_B_PER_W = N_ID // _NW
_CHUNK = 128
_N_CHUNKS = _B_PER_W // _CHUNK


def _sc_gather_body(n_id_hbm, mem_hbm, lu_hbm, h_out, lu_out,
                    idx_v, rows_v, lu_v, sem, sem_lu):
    wid = lax.axis_index("s") * _NC + lax.axis_index("c")
    base = wid * _B_PER_W
    pltpu.sync_copy(n_id_hbm.at[pl.ds(base, _B_PER_W)], idx_v)
    row_cps = []
    lu_cps = []
    for c in range(_N_CHUNKS):
        idx_c = idx_v.at[pl.ds(c * _CHUNK, _CHUNK)]
        row_cps.append(pltpu.async_copy(mem_hbm.at[idx_c], rows_v.at[c], sem))
        lu_cps.append(pltpu.async_copy(lu_hbm.at[idx_c], lu_v.at[c], sem_lu))
    for c in range(_N_CHUNKS):
        off = base + c * _CHUNK
        row_cps[c].wait()
        pltpu.sync_copy(rows_v.at[c], h_out.at[pl.ds(off, _CHUNK)])
        lu_cps[c].wait()
        pltpu.sync_copy(lu_v.at[c], lu_out.at[pl.ds(off, _CHUNK)])


_sc_gather = functools.partial(
    pl.kernel,
    mesh=plsc.VectorSubcoreMesh(core_axis_name="c", subcore_axis_name="s"),
    out_type=[
        jax.ShapeDtypeStruct((N_ID, MEM_DIM), jnp.float32),
        jax.ShapeDtypeStruct((N_ID,), jnp.int32),
    ],
    scratch_types=[
        pltpu.VMEM((_B_PER_W,), jnp.int32),
        pltpu.VMEM((_N_CHUNKS, _CHUNK, MEM_DIM), jnp.float32),
        pltpu.VMEM((_N_CHUNKS, _CHUNK), jnp.int32),
        pltpu.SemaphoreType.DMA,
        pltpu.SemaphoreType.DMA,
    ],
)(_sc_gather_body)


_BLK = 4096


def _gru_body(h_ref, wr_ref, wz_ref, wn_ref, br_ref, bz_ref, bni_ref, bnh_ref,
              out_ref):
    h = h_ref[...]
    dn = (((1,), (0,)), ((), ()))
    r = jax.nn.sigmoid(
        jax.lax.dot_general(h, wr_ref[...], dn,
                            preferred_element_type=jnp.float32) + br_ref[...])
    z = jax.nn.sigmoid(
        jax.lax.dot_general(h, wz_ref[...], dn,
                            preferred_element_type=jnp.float32) + bz_ref[...])
    ghn = jax.lax.dot_general(h, wn_ref[...], dn,
                              preferred_element_type=jnp.float32) + bnh_ref[...]
    n = jnp.tanh(bni_ref[...] + r * ghn)
    out_ref[...] = n + z * (h - n)


def _gru(h, wr, wz, wn, br, bz, bni, bnh):
    grid = N_ID // _BLK
    w_spec = pl.BlockSpec((MEM_DIM, MEM_DIM), lambda i: (0, 0))
    b_spec = pl.BlockSpec((1, MEM_DIM), lambda i: (0, 0))
    return pl.pallas_call(
        _gru_body,
        grid=(grid,),
        in_specs=[
            pl.BlockSpec((_BLK, MEM_DIM), lambda i: (i, 0)),
            w_spec, w_spec, w_spec,
            b_spec, b_spec, b_spec, b_spec,
        ],
        out_specs=pl.BlockSpec((_BLK, MEM_DIM), lambda i: (i, 0)),
        out_shape=jax.ShapeDtypeStruct((N_ID, MEM_DIM), jnp.float32),
    )(h, wr, wz, wn, br, bz, bni, bnh)


@jax.jit
def kernel(n_id, memory, last_update, W_ih, W_hh, b_ih, b_hh):
    del W_ih
    h, lu_out = _sc_gather(n_id.astype(jnp.int32), memory,
                           last_update.astype(jnp.int32))
    D = MEM_DIM
    wr = W_hh[:D].T
    wz = W_hh[D:2 * D].T
    wn = W_hh[2 * D:].T
    br = (b_ih[:D] + b_hh[:D]).reshape(1, D)
    bz = (b_ih[D:2 * D] + b_hh[D:2 * D]).reshape(1, D)
    bni = b_ih[2 * D:].reshape(1, D)
    bnh = b_hh[2 * D:].reshape(1, D)
    mem_out = _gru(h, wr, wz, wn, br, bz, bni, bnh)
    return (mem_out, lu_out.astype(last_update.dtype))
